# baseline (speedup 1.0000x reference)
import functools

import jax
import jax.numpy as jnp
from jax import lax
from jax.experimental import pallas as pl
from jax.experimental.pallas import tpu as pltpu
from jax.experimental.pallas import tpu_sc as plsc

N = 10000
D = 128
NC = 2
NS = 16
L = 16
R = 10240
DUM = 10000
WS = D // NC
CH = 128
RPT = R // NS
CCH = 40

EM = 320000
EK = 160000
CPT_M = 8 * ((EM + NS * CH * 8 - 1) // (NS * CH * 8))
CPT_K = 8 * ((EK + NS * CH * 8 - 1) // (NS * CH * 8))
ROWS_M = NS * CPT_M
ROWS_K = NS * CPT_K


@functools.cache
def _mesh():
  return plsc.VectorSubcoreMesh(
      core_axis_name="c", subcore_axis_name="s", num_cores=NC, num_subcores=NS)


def _fill(buf, rows, width, value):
  v = jnp.full((L,), value, jnp.float32)
  for i in range(rows):
    for f in range(width // L):
      buf[i, pl.ds(f * L, L)] = v


NSLOT = 4


def _scatter_step(u_ref, pk_e, cpt, acc, pbs, gidxs, sidxs, rbs, sems,
                  base, ebase, hbm_row):
  def istart(c, b):
    pltpu.async_copy(pk_e.at[pl.ds(ebase + c * CH, CH)], pbs[b], sems[b])

  def iw(b):
    pltpu.make_async_copy(pk_e.at[pl.ds(0, CH)], pbs[b], sems[b]).wait()

  def unpack(b):
    for j in range(CH // L):
      w = pbs[b][pl.ds(j * L, L)]
      s = w & jnp.int32(0xFFFF)
      dd = w >> 16
      gidxs[b][pl.ds(j * L, L)] = s + base
      sidxs[b][pl.ds(j * L, L)] = jnp.where(s == dd, jnp.int32(DUM), dd)

  def gstart(b):
    pltpu.async_copy(u_ref.at[gidxs[b]], rbs[b], sems[b])

  def dwait(b):
    pltpu.make_async_copy(hbm_row, rbs[b], sems[b]).wait()

  def sstart(b):
    pltpu.async_copy(rbs[b], acc.at[sidxs[b]], sems[b], add=True)

  for b in range(NSLOT):
    istart(b, b)
  for b in range(NSLOT):
    iw(b)
    unpack(b)
    gstart(b)

  def body(i, carry):
    c0 = NSLOT * i
    for b in range(NSLOT):
      dwait(b)
      sstart(b)
    for b in range(NSLOT):
      dwait(b)
      istart(c0 + NSLOT + b, b)
    for b in range(NSLOT):
      iw(b)
      unpack(b)
      gstart(b)
    return carry

  lax.fori_loop(0, cpt // NSLOT - 1, body, 0)
  for b in range(NSLOT):
    dwait(b)
    sstart(b)
  for b in range(NSLOT):
    dwait(b)


def _combine(src_ref, dst_ref, z_ref, scale_ref, acc, abuf, ubuf, obuf, scb,
             zcbuf, zbuf, src_off, dst_off, sub, rezero):
  nrow0 = sub * RPT

  def body(i, carry):
    r0 = nrow0 + i * CCH
    pltpu.sync_copy(acc.at[pl.ds(r0, CCH)], abuf)
    pltpu.sync_copy(src_ref.at[pl.ds(src_off + r0, CCH)], ubuf)
    pltpu.sync_copy(scale_ref.at[pl.ds(r0, CCH)], scb)
    if rezero:
      pltpu.sync_copy(zbuf, acc.at[pl.ds(r0, CCH)])
    if z_ref is not None:
      pltpu.sync_copy(z_ref.at[pl.ds(dst_off + r0, CCH)], zcbuf)

    def rbody(r, c):
      sc = scb[r, :]
      for f in range(WS // L):
        ov = sc * (abuf[r, pl.ds(f * L, L)] + ubuf[r, pl.ds(f * L, L)])
        if z_ref is not None:
          ov = ov + zcbuf[r, pl.ds(f * L, L)]
        obuf[r, pl.ds(f * L, L)] = ov
      return c

    lax.fori_loop(0, CCH, rbody, 0)
    pltpu.sync_copy(obuf, dst_ref.at[pl.ds(dst_off + r0, CCH)])
    return carry

  lax.fori_loop(0, RPT // CCH, body, 0)


def _make_chain(K, cpt, with_z):
  def body(u0, pk_e, d2, dinv, *rest):
    if with_z:
      z0, out = rest[0], rest[1]
      scratches = rest[2:]
    else:
      z0 = None
      out = rest[0]
      scratches = rest[1:]
    scratches = list(scratches)
    acc = scratches.pop(0)
    ua = scratches.pop(0) if K > 1 else None
    pbs = [scratches.pop(0) for _ in range(NSLOT)]
    gidxs = [scratches.pop(0) for _ in range(NSLOT)]
    sidxs = [scratches.pop(0) for _ in range(NSLOT)]
    rbs = [scratches.pop(0) for _ in range(NSLOT)]
    abuf, ubuf, obuf, scb, zbuf = scratches[:5]
    zcbuf = scratches[5] if with_z else None
    sems = scratches[-NSLOT:]
    core = lax.axis_index("c")
    sub = lax.axis_index("s")
    base = core * R
    nrow0 = sub * RPT
    ebase = sub * cpt * CH
    hbm_row = out.at[pl.ds(0, CH)]
    _fill(zbuf, CCH, WS, 0.0)

    for i in range(RPT // CCH):
      pltpu.sync_copy(zbuf, acc.at[pl.ds(nrow0 + i * CCH, CCH)])

    if K == 1:
      plsc.subcore_barrier()
      _scatter_step(u0, pk_e, cpt, acc, pbs, gidxs, sidxs, rbs, sems,
                    base, ebase, hbm_row)
      plsc.subcore_barrier()
      _combine(u0, out, z0, dinv, acc, abuf, ubuf, obuf, scb, zcbuf, zbuf,
               base, base, sub, False)
    else:
      pltpu.sync_copy(u0.at[pl.ds(base + nrow0, RPT)],
                      ua.at[pl.ds(nrow0, RPT)])

      def step(i, carry):
        plsc.subcore_barrier()
        _scatter_step(ua, pk_e, cpt, acc, pbs, gidxs, sidxs, rbs, sems,
                      0, ebase, hbm_row)
        plsc.subcore_barrier()
        _combine(ua, ua, None, d2, acc, abuf, ubuf, obuf, scb, None, zbuf,
                 0, 0, sub, True)
        return carry

      lax.fori_loop(0, K - 1, step, 0)
      plsc.subcore_barrier()
      _scatter_step(ua, pk_e, cpt, acc, pbs, gidxs, sidxs, rbs, sems,
                    0, ebase, hbm_row)
      plsc.subcore_barrier()
      _combine(ua, out, None, dinv, acc, abuf, ubuf, obuf, scb, None, zbuf,
               0, base, sub, False)

  out_t = jax.ShapeDtypeStruct((NC * R, WS), jnp.float32)
  scratch = [pltpu.VMEM_SHARED((R, WS), jnp.float32)]
  if K > 1:
    scratch += [pltpu.VMEM_SHARED((R, WS), jnp.float32)]
  scratch += [pltpu.VMEM((CH,), jnp.int32) for _ in range(3 * NSLOT)]
  scratch += [pltpu.VMEM((CH, WS), jnp.float32) for _ in range(NSLOT)]
  scratch += [
      pltpu.VMEM((CCH, WS), jnp.float32),
      pltpu.VMEM((CCH, WS), jnp.float32),
      pltpu.VMEM((CCH, WS), jnp.float32),
      pltpu.VMEM((CCH, L), jnp.float32),
      pltpu.VMEM((CCH, WS), jnp.float32),
  ]
  if with_z:
    scratch += [pltpu.VMEM((CCH, WS), jnp.float32)]
  scratch += [pltpu.SemaphoreType.DMA for _ in range(NSLOT)]
  return pl.kernel(body, out_type=out_t, mesh=_mesh(), scratch_types=scratch,
                   compiler_params=pltpu.CompilerParams(
                       use_tc_tiling_on_sc=False))


def _deg_body(pk_m, pk_k, deg_m, deg_k, acc, pball, dball, onesb, abuf, zbuf,
              sem):
  core = lax.axis_index("c")
  sub = lax.axis_index("s")
  _fill(zbuf, CCH, L, 0.0)
  _fill(onesb, CH, L, 1.0)

  def histo(pk_ref, cpt, deg_out):
    nrow0 = sub * RPT
    pltpu.sync_copy(pk_ref.at[pl.ds(sub * cpt * CH, cpt * CH)],
                    pball.at[pl.ds(0, cpt * CH)])
    for i in range(RPT // CCH):
      pltpu.sync_copy(zbuf, acc.at[pl.ds(nrow0 + i * CCH, CCH)])

    def prep(i, c):
      for j in range(CH // L):
        w = pball[pl.ds(i * CH + j * L, L)]
        s = w & jnp.int32(0xFFFF)
        dd = w >> 16
        dball[i, pl.ds(j * L, L)] = jnp.where(s == dd, jnp.int32(DUM), dd)
      return c

    lax.fori_loop(0, cpt, prep, 0)
    plsc.subcore_barrier()

    def group(i, c):
      for j in range(8):
        pltpu.async_copy(onesb, acc.at[dball.at[8 * i + j]], sem, add=True)
      for j in range(8):
        pltpu.make_async_copy(deg_out.at[pl.ds(0, CH)], onesb, sem).wait()
      return c

    lax.fori_loop(0, cpt // 8, group, 0)
    plsc.subcore_barrier()

    def comb(i, c):
      r0 = nrow0 + i * CCH
      pltpu.sync_copy(acc.at[pl.ds(r0, CCH)], abuf)

      def rbody(r, cc):
        abuf[r, :] = abuf[r, :] + 1.0
        return cc

      lax.fori_loop(0, CCH, rbody, 0)
      pltpu.sync_copy(abuf, deg_out.at[pl.ds(r0, CCH)])
      return c

    lax.fori_loop(0, RPT // CCH, comb, 0)

  @pl.when(core == 0)
  def _():
    histo(pk_m, CPT_M, deg_m)

  @pl.when(core == 1)
  def _():
    histo(pk_k, CPT_K, deg_k)


@functools.cache
def _deg_kernel():
  return pl.kernel(
    _deg_body,
    out_type=(jax.ShapeDtypeStruct((R, L), jnp.float32),) * 2,
    mesh=_mesh(),
    scratch_types=[
        pltpu.VMEM_SHARED((R, L), jnp.float32),
        pltpu.VMEM((CPT_M * CH,), jnp.int32),
        pltpu.VMEM((CPT_M, CH), jnp.int32),
        pltpu.VMEM((CH, L), jnp.float32),
        pltpu.VMEM((CCH, L), jnp.float32),
        pltpu.VMEM((CCH, L), jnp.float32),
        pltpu.SemaphoreType.DMA,
    ],
    compiler_params=pltpu.CompilerParams(use_tc_tiling_on_sc=False),
  )


def _norm_body(dm_ref, dk_ref, im_ref, qm_ref, ik_ref, qk_ref):
  dm = dm_ref[...]
  dk = dk_ref[...]
  im_ref[...] = lax.rsqrt(dm)
  qm_ref[...] = 1.0 / dm
  ik_ref[...] = lax.rsqrt(dk)
  qk_ref[...] = 1.0 / dk


def _norm(deg_m, deg_k):
  return pl.pallas_call(
      _norm_body,
      out_shape=(jax.ShapeDtypeStruct((R, L), jnp.float32),) * 4,
  )(deg_m, deg_k)


def _mm_body(x_ref, w_ref, b_ref, dm_ref, dk_ref, o_ref):
  y = jnp.dot(x_ref[...], w_ref[...], preferred_element_type=jnp.float32)
  y = y + b_ref[0:1, :]
  s = dm_ref[:, 0:1]
  sk = dk_ref[:, 0:1]
  o_ref[:, 0:384] = y[:, 0:384] * s
  o_ref[:, 384:512] = y[:, 384:512] * sk


_MMB = 1000


def _matmul(x, wt, b8, dm, dk):
  return pl.pallas_call(
      _mm_body,
      grid=(N // _MMB,),
      in_specs=[
          pl.BlockSpec((_MMB, D), lambda i: (i, 0)),
          pl.BlockSpec((D, 512), lambda i: (0, 0)),
          pl.BlockSpec((8, 512), lambda i: (0, 0)),
          pl.BlockSpec((_MMB, L), lambda i: (i, 0)),
          pl.BlockSpec((_MMB, L), lambda i: (i, 0)),
      ],
      out_specs=pl.BlockSpec((_MMB, 512), lambda i: (i, 0)),
      out_shape=jax.ShapeDtypeStruct((N, 512), jnp.float32),
  )(x, wt, b8, dm, dk)


def _pack_edges(ei, rows):
  e = ei.shape[1]
  tot = rows * CH
  src = jnp.pad(ei[0].astype(jnp.int32), (0, tot - e))
  pad_d = DUM + (jnp.arange(tot - e, dtype=jnp.int32) % (R - DUM))
  dst = jnp.concatenate([ei[1].astype(jnp.int32), pad_d])
  return src | (dst << 16)


def _to_sc(y):
  lo = jnp.pad(y[:, 0:WS], ((0, R - N), (0, 0)))
  hi = jnp.pad(y[:, WS:D], ((0, R - N), (0, 0)))
  return jnp.concatenate([lo, hi], axis=0)


def _from_sc(u):
  return jnp.concatenate([u[0:N], u[R:R + N]], axis=1)


_chain2 = functools.cache(lambda: _make_chain(2, CPT_M, False))
_chain10 = functools.cache(lambda: _make_chain(10, CPT_M, False))
_chain1z = functools.cache(lambda: _make_chain(1, CPT_K, True))


def kernel(x, edge_index, knn_graph, W_s1, b_s1, W_s2, b_s2, W_f2, b_f2):
  pk_m = _pack_edges(edge_index, ROWS_M)
  pk_k = _pack_edges(knn_graph, ROWS_K)

  deg_m, deg_k = _deg_kernel()(pk_m, pk_k)
  dinv_m, d2_m, dinv_k, d2_k = _norm(deg_m, deg_k)

  wt = jnp.concatenate([W_s1, W_s2, W_f2, W_f2], axis=0).T
  bcat = jnp.concatenate([b_s1, b_s2, b_f2, b_f2])
  b8 = jnp.broadcast_to(bcat[None, :], (8, 512))
  y = _matmul(x, wt, b8, dinv_m[0:N], dinv_k[0:N])

  u_s1 = _to_sc(y[:, 0:128])
  u_s2 = _to_sc(y[:, 128:256])
  u_f = _to_sc(y[:, 256:384])
  u_k = _to_sc(y[:, 384:512])

  h0u = _chain2()(u_s1, pk_m, d2_m, dinv_m)
  h1u = _chain10()(u_s2, pk_m, d2_m, dinv_m)
  z0u = _chain2()(u_f, pk_m, d2_m, dinv_m)
  z1u = _chain1z()(u_k, pk_k, d2_k, dinv_k, z0u)

  return (_from_sc(h0u), _from_sc(h1u), _from_sc(z0u), _from_sc(z1u))


# speedup vs baseline: 1.1193x; 1.1193x over previous
import functools

import jax
import jax.numpy as jnp
from jax import lax
from jax.experimental import pallas as pl
from jax.experimental.pallas import tpu as pltpu
from jax.experimental.pallas import tpu_sc as plsc

N = 10000
D = 128
NC = 2
NS = 16
L = 16
R = 10240
DUM = 10000
WS = D // NC
CH = 128
RPT = R // NS
CCH = 40

EM = 320000
EK = 160000
CPT_M = 8 * ((EM + NS * CH * 8 - 1) // (NS * CH * 8))
CPT_K = 8 * ((EK + NS * CH * 8 - 1) // (NS * CH * 8))
ROWS_M = NS * CPT_M
ROWS_K = NS * CPT_K


@functools.cache
def _mesh():
  return plsc.VectorSubcoreMesh(
      core_axis_name="c", subcore_axis_name="s", num_cores=NC, num_subcores=NS)


def _fill(buf, rows, width, value):
  v = jnp.full((L,), value, jnp.float32)
  for i in range(rows):
    for f in range(width // L):
      buf[i, pl.ds(f * L, L)] = v


NSLOT = 4
CE = 64


def _scatter_step(u_ref, cpt, acc, pball, gidxs, sidxs, rbs, sems,
                  base, hbm_row):
  def unpack(c, b):
    for j in range(CE // L):
      w = pball[pl.ds(c * CE + j * L, L)]
      s = w & jnp.int32(0xFFFF)
      dd = w >> 16
      gidxs[b][pl.ds(j * L, L)] = s + base
      sidxs[b][pl.ds(j * L, L)] = jnp.where(s == dd, jnp.int32(DUM), dd)

  def gstart(b):
    pltpu.async_copy(u_ref.at[gidxs[b]], rbs[b], sems[b])

  def dwait(b):
    pltpu.make_async_copy(hbm_row, rbs[b], sems[b]).wait()

  def sstart(b):
    pltpu.async_copy(rbs[b], acc.at[sidxs[b]], sems[b], add=True)

  for b in range(NSLOT):
    unpack(b, b)
    gstart(b)

  def body(i, carry):
    c0 = NSLOT * i
    for b in range(NSLOT):
      dwait(b)
      sstart(b)
    for b in range(NSLOT):
      dwait(b)
      unpack(c0 + NSLOT + b, b)
      gstart(b)
    return carry

  lax.fori_loop(0, cpt // NSLOT - 1, body, 0)
  for b in range(NSLOT):
    dwait(b)
    sstart(b)
  for b in range(NSLOT):
    dwait(b)


def _combine(src_ref, dst_ref, z_ref, scale_ref, acc, abuf, ubuf, obuf, scb,
             zcbuf, zbuf, src_off, dst_off, sub, rezero):
  nrow0 = sub * RPT

  def body(i, carry):
    r0 = nrow0 + i * CCH
    pltpu.sync_copy(acc.at[pl.ds(r0, CCH)], abuf)
    pltpu.sync_copy(src_ref.at[pl.ds(src_off + r0, CCH)], ubuf)
    pltpu.sync_copy(scale_ref.at[pl.ds(r0, CCH)], scb)
    if rezero:
      pltpu.sync_copy(zbuf, acc.at[pl.ds(r0, CCH)])
    if z_ref is not None:
      pltpu.sync_copy(z_ref.at[pl.ds(dst_off + r0, CCH)], zcbuf)

    def rbody(r, c):
      sc = scb[r, :]
      for f in range(WS // L):
        ov = sc * (abuf[r, pl.ds(f * L, L)] + ubuf[r, pl.ds(f * L, L)])
        if z_ref is not None:
          ov = ov + zcbuf[r, pl.ds(f * L, L)]
        obuf[r, pl.ds(f * L, L)] = ov
      return c

    lax.fori_loop(0, CCH, rbody, 0)
    pltpu.sync_copy(obuf, dst_ref.at[pl.ds(dst_off + r0, CCH)])
    return carry

  lax.fori_loop(0, RPT // CCH, body, 0)


def _make_chain(K, ept, with_z):
  cpt = ept // CE

  def body(u0, pk_e, d2, dinv, *rest):
    if with_z:
      z0, out = rest[0], rest[1]
      scratches = rest[2:]
    else:
      z0 = None
      out = rest[0]
      scratches = rest[1:]
    scratches = list(scratches)
    acc = scratches.pop(0)
    ua = scratches.pop(0) if K > 1 else None
    pball = scratches.pop(0)
    gidxs = [scratches.pop(0) for _ in range(NSLOT)]
    sidxs = [scratches.pop(0) for _ in range(NSLOT)]
    rbs = [scratches.pop(0) for _ in range(NSLOT)]
    abuf, ubuf, obuf, scb, zbuf = scratches[:5]
    zcbuf = scratches[5] if with_z else None
    sems = scratches[-NSLOT:]
    core = lax.axis_index("c")
    sub = lax.axis_index("s")
    base = core * R
    nrow0 = sub * RPT
    hbm_row = out.at[pl.ds(0, CE)]
    _fill(zbuf, CCH, WS, 0.0)
    pltpu.sync_copy(pk_e.at[pl.ds(sub * ept, ept)], pball)

    for i in range(RPT // CCH):
      pltpu.sync_copy(zbuf, acc.at[pl.ds(nrow0 + i * CCH, CCH)])

    if K == 1:
      plsc.subcore_barrier()
      _scatter_step(u0, cpt, acc, pball, gidxs, sidxs, rbs, sems,
                    base, hbm_row)
      plsc.subcore_barrier()
      _combine(u0, out, z0, dinv, acc, abuf, ubuf, obuf, scb, zcbuf, zbuf,
               base, base, sub, False)
    else:
      pltpu.sync_copy(u0.at[pl.ds(base + nrow0, RPT)],
                      ua.at[pl.ds(nrow0, RPT)])

      def step(i, carry):
        plsc.subcore_barrier()
        _scatter_step(ua, cpt, acc, pball, gidxs, sidxs, rbs, sems,
                      0, hbm_row)
        plsc.subcore_barrier()
        _combine(ua, ua, None, d2, acc, abuf, ubuf, obuf, scb, None, zbuf,
                 0, 0, sub, True)
        return carry

      lax.fori_loop(0, K - 1, step, 0)
      plsc.subcore_barrier()
      _scatter_step(ua, cpt, acc, pball, gidxs, sidxs, rbs, sems,
                    0, hbm_row)
      plsc.subcore_barrier()
      _combine(ua, out, None, dinv, acc, abuf, ubuf, obuf, scb, None, zbuf,
               0, base, sub, False)

  out_t = jax.ShapeDtypeStruct((NC * R, WS), jnp.float32)
  scratch = [pltpu.VMEM_SHARED((R, WS), jnp.float32)]
  if K > 1:
    scratch += [pltpu.VMEM_SHARED((R, WS), jnp.float32)]
  scratch += [pltpu.VMEM((ept,), jnp.int32)]
  scratch += [pltpu.VMEM((CE,), jnp.int32) for _ in range(2 * NSLOT)]
  scratch += [pltpu.VMEM((CE, WS), jnp.float32) for _ in range(NSLOT)]
  scratch += [
      pltpu.VMEM((CCH, WS), jnp.float32),
      pltpu.VMEM((CCH, WS), jnp.float32),
      pltpu.VMEM((CCH, WS), jnp.float32),
      pltpu.VMEM((CCH, L), jnp.float32),
      pltpu.VMEM((CCH, WS), jnp.float32),
  ]
  if with_z:
    scratch += [pltpu.VMEM((CCH, WS), jnp.float32)]
  scratch += [pltpu.SemaphoreType.DMA for _ in range(NSLOT)]
  return pl.kernel(body, out_type=out_t, mesh=_mesh(), scratch_types=scratch,
                   compiler_params=pltpu.CompilerParams(
                       use_tc_tiling_on_sc=False))


def _deg_body(pk_m, pk_k, deg_m, deg_k, acc, pball, dball, onesb, abuf, zbuf,
              sem):
  core = lax.axis_index("c")
  sub = lax.axis_index("s")
  _fill(zbuf, CCH, L, 0.0)
  _fill(onesb, CH, L, 1.0)

  def histo(pk_ref, cpt, deg_out):
    nrow0 = sub * RPT
    pltpu.sync_copy(pk_ref.at[pl.ds(sub * cpt * CH, cpt * CH)],
                    pball.at[pl.ds(0, cpt * CH)])
    for i in range(RPT // CCH):
      pltpu.sync_copy(zbuf, acc.at[pl.ds(nrow0 + i * CCH, CCH)])

    def prep(i, c):
      for j in range(CH // L):
        w = pball[pl.ds(i * CH + j * L, L)]
        s = w & jnp.int32(0xFFFF)
        dd = w >> 16
        dball[i, pl.ds(j * L, L)] = jnp.where(s == dd, jnp.int32(DUM), dd)
      return c

    lax.fori_loop(0, cpt, prep, 0)
    plsc.subcore_barrier()

    def group(i, c):
      for j in range(8):
        pltpu.async_copy(onesb, acc.at[dball.at[8 * i + j]], sem, add=True)
      for j in range(8):
        pltpu.make_async_copy(deg_out.at[pl.ds(0, CH)], onesb, sem).wait()
      return c

    lax.fori_loop(0, cpt // 8, group, 0)
    plsc.subcore_barrier()

    def comb(i, c):
      r0 = nrow0 + i * CCH
      pltpu.sync_copy(acc.at[pl.ds(r0, CCH)], abuf)

      def rbody(r, cc):
        abuf[r, :] = abuf[r, :] + 1.0
        return cc

      lax.fori_loop(0, CCH, rbody, 0)
      pltpu.sync_copy(abuf, deg_out.at[pl.ds(r0, CCH)])
      return c

    lax.fori_loop(0, RPT // CCH, comb, 0)

  @pl.when(core == 0)
  def _():
    histo(pk_m, CPT_M, deg_m)

  @pl.when(core == 1)
  def _():
    histo(pk_k, CPT_K, deg_k)


@functools.cache
def _deg_kernel():
  return pl.kernel(
    _deg_body,
    out_type=(jax.ShapeDtypeStruct((R, L), jnp.float32),) * 2,
    mesh=_mesh(),
    scratch_types=[
        pltpu.VMEM_SHARED((R, L), jnp.float32),
        pltpu.VMEM((CPT_M * CH,), jnp.int32),
        pltpu.VMEM((CPT_M, CH), jnp.int32),
        pltpu.VMEM((CH, L), jnp.float32),
        pltpu.VMEM((CCH, L), jnp.float32),
        pltpu.VMEM((CCH, L), jnp.float32),
        pltpu.SemaphoreType.DMA,
    ],
    compiler_params=pltpu.CompilerParams(use_tc_tiling_on_sc=False),
  )


def _norm_body(dm_ref, dk_ref, im_ref, qm_ref, ik_ref, qk_ref):
  dm = dm_ref[...]
  dk = dk_ref[...]
  im_ref[...] = lax.rsqrt(dm)
  qm_ref[...] = 1.0 / dm
  ik_ref[...] = lax.rsqrt(dk)
  qk_ref[...] = 1.0 / dk


def _norm(deg_m, deg_k):
  return pl.pallas_call(
      _norm_body,
      out_shape=(jax.ShapeDtypeStruct((R, L), jnp.float32),) * 4,
  )(deg_m, deg_k)


def _mm_body(x_ref, w_ref, b_ref, dm_ref, dk_ref, o_ref):
  y = jnp.dot(x_ref[...], w_ref[...], preferred_element_type=jnp.float32)
  y = y + b_ref[0:1, :]
  s = dm_ref[:, 0:1]
  sk = dk_ref[:, 0:1]
  o_ref[:, 0:384] = y[:, 0:384] * s
  o_ref[:, 384:512] = y[:, 384:512] * sk


_MMB = 1000


def _matmul(x, wt, b8, dm, dk):
  return pl.pallas_call(
      _mm_body,
      grid=(N // _MMB,),
      in_specs=[
          pl.BlockSpec((_MMB, D), lambda i: (i, 0)),
          pl.BlockSpec((D, 512), lambda i: (0, 0)),
          pl.BlockSpec((8, 512), lambda i: (0, 0)),
          pl.BlockSpec((_MMB, L), lambda i: (i, 0)),
          pl.BlockSpec((_MMB, L), lambda i: (i, 0)),
      ],
      out_specs=pl.BlockSpec((_MMB, 512), lambda i: (i, 0)),
      out_shape=jax.ShapeDtypeStruct((N, 512), jnp.float32),
  )(x, wt, b8, dm, dk)


def _pack_edges(ei, rows):
  e = ei.shape[1]
  tot = rows * CH
  src = jnp.pad(ei[0].astype(jnp.int32), (0, tot - e))
  pad_d = DUM + (jnp.arange(tot - e, dtype=jnp.int32) % (R - DUM))
  dst = jnp.concatenate([ei[1].astype(jnp.int32), pad_d])
  return src | (dst << 16)


def _to_sc(y):
  lo = jnp.pad(y[:, 0:WS], ((0, R - N), (0, 0)))
  hi = jnp.pad(y[:, WS:D], ((0, R - N), (0, 0)))
  return jnp.concatenate([lo, hi], axis=0)


def _from_sc(u):
  return jnp.concatenate([u[0:N], u[R:R + N]], axis=1)


EPT_M = ROWS_M * CH // NS
EPT_K = ROWS_K * CH // NS

_chain2 = functools.cache(lambda: _make_chain(2, EPT_M, False))
_chain10 = functools.cache(lambda: _make_chain(10, EPT_M, False))
_chain1z = functools.cache(lambda: _make_chain(1, EPT_K, True))


def kernel(x, edge_index, knn_graph, W_s1, b_s1, W_s2, b_s2, W_f2, b_f2):
  pk_m = _pack_edges(edge_index, ROWS_M)
  pk_k = _pack_edges(knn_graph, ROWS_K)

  deg_m, deg_k = _deg_kernel()(pk_m, pk_k)
  dinv_m, d2_m, dinv_k, d2_k = _norm(deg_m, deg_k)

  wt = jnp.concatenate([W_s1, W_s2, W_f2, W_f2], axis=0).T
  bcat = jnp.concatenate([b_s1, b_s2, b_f2, b_f2])
  b8 = jnp.broadcast_to(bcat[None, :], (8, 512))
  y = _matmul(x, wt, b8, dinv_m[0:N], dinv_k[0:N])

  u_s1 = _to_sc(y[:, 0:128])
  u_s2 = _to_sc(y[:, 128:256])
  u_f = _to_sc(y[:, 256:384])
  u_k = _to_sc(y[:, 384:512])

  h0u = _chain2()(u_s1, pk_m, d2_m, dinv_m)
  h1u = _chain10()(u_s2, pk_m, d2_m, dinv_m)
  z0u = _chain2()(u_f, pk_m, d2_m, dinv_m)
  z1u = _chain1z()(u_k, pk_k, d2_k, dinv_k, z0u)

  return (_from_sc(h0u), _from_sc(h1u), _from_sc(z0u), _from_sc(z1u))


# speedup vs baseline: 1.2040x; 1.0756x over previous
import functools

import jax
import jax.numpy as jnp
from jax import lax
from jax.experimental import pallas as pl
from jax.experimental.pallas import tpu as pltpu
from jax.experimental.pallas import tpu_sc as plsc

N = 10000
D = 128
NC = 2
NS = 16
L = 16
R = 10240
DUM = 10000
WS = D // NC
CH = 128
RPT = R // NS
CCH = 40

EM = 320000 + N
EK = 160000 + N
EPT_M = CH * (2 * ((EM + NS * CH * 2 - 1) // (NS * CH * 2)))
EPT_K = CH * (2 * ((EK + NS * CH * 2 - 1) // (NS * CH * 2)))
CPT_M = EPT_M // CH
CPT_K = EPT_K // CH


@functools.cache
def _mesh():
  return plsc.VectorSubcoreMesh(
      core_axis_name="c", subcore_axis_name="s", num_cores=NC, num_subcores=NS)


def _fill(buf, rows, width, value):
  v = jnp.full((L,), value, jnp.float32)
  for i in range(rows):
    for f in range(width // L):
      buf[i, pl.ds(f * L, L)] = v


def _scatter_step(u_ref, cpt, acc, pball, gidx0, sidx0, gidx1, sidx1,
                  rb0, rb1, sem0, sem1, base, hbm_row):
  def unpack(c, gidx, sidx):
    for j in range(CH // L):
      w = pball[pl.ds(c * CH + j * L, L)]
      gidx[pl.ds(j * L, L)] = (w & jnp.int32(0xFFFF)) + base
      sidx[pl.ds(j * L, L)] = w >> 16

  def g(c, gidx, sidx, rb, sem):
    unpack(c, gidx, sidx)
    pltpu.async_copy(u_ref.at[gidx], rb, sem)

  def gw(rb, sem):
    pltpu.make_async_copy(hbm_row, rb, sem).wait()

  def s(sidx, rb):
    pltpu.sync_copy(rb, acc.at[sidx], add=True)

  g(0, gidx0, sidx0, rb0, sem0)
  g(1, gidx1, sidx1, rb1, sem1)

  def body(i, carry):
    c = 2 * i
    gw(rb0, sem0)
    s(sidx0, rb0)
    g(c + 2, gidx0, sidx0, rb0, sem0)
    gw(rb1, sem1)
    s(sidx1, rb1)
    g(c + 3, gidx1, sidx1, rb1, sem1)
    return carry

  lax.fori_loop(0, (cpt - 2) // 2, body, 0)
  gw(rb0, sem0)
  s(sidx0, rb0)
  gw(rb1, sem1)
  s(sidx1, rb1)


def _combine(dst_ref, z_ref, scale_ref, acc, abuf, obuf, scb, zcbuf, zbuf,
             dst_off, sub, rezero):
  nrow0 = sub * RPT

  def body(i, carry):
    r0 = nrow0 + i * CCH
    pltpu.sync_copy(acc.at[pl.ds(r0, CCH)], abuf)
    pltpu.sync_copy(scale_ref.at[pl.ds(r0, CCH)], scb)
    if rezero:
      pltpu.sync_copy(zbuf, acc.at[pl.ds(r0, CCH)])
    if z_ref is not None:
      pltpu.sync_copy(z_ref.at[pl.ds(dst_off + r0, CCH)], zcbuf)

    def rbody(r, c):
      sc = scb[r, :]
      for f in range(WS // L):
        ov = sc * abuf[r, pl.ds(f * L, L)]
        if z_ref is not None:
          ov = ov + zcbuf[r, pl.ds(f * L, L)]
        obuf[r, pl.ds(f * L, L)] = ov
      return c

    lax.fori_loop(0, CCH, rbody, 0)
    pltpu.sync_copy(obuf, dst_ref.at[pl.ds(dst_off + r0, CCH)])
    return carry

  lax.fori_loop(0, RPT // CCH, body, 0)


def _prop_body(u_s1, u_s2, u_f, u_k, pk_m, pk_k, d2, dinv, dinv_k,
               h0, h1, z0, z1, acc, ua, pball, gidx0, sidx0, gidx1, sidx1,
               rb0, rb1, abuf, obuf, scb, zcbuf, zbuf, sem0, sem1):
  core = lax.axis_index("c")
  sub = lax.axis_index("s")
  base = core * R
  nrow0 = sub * RPT
  hbm_row = h0.at[pl.ds(0, CH)]
  _fill(zbuf, CCH, WS, 0.0)
  pltpu.sync_copy(pk_m.at[pl.ds(sub * EPT_M, EPT_M)], pball)

  for i in range(RPT // CCH):
    pltpu.sync_copy(zbuf, acc.at[pl.ds(nrow0 + i * CCH, CCH)])

  def chain(u0, k, out):
    pltpu.sync_copy(u0.at[pl.ds(base + nrow0, RPT)], ua.at[pl.ds(nrow0, RPT)])

    def step(i, carry):
      plsc.subcore_barrier()
      _scatter_step(ua, CPT_M, acc, pball, gidx0, sidx0, gidx1, sidx1,
                    rb0, rb1, sem0, sem1, 0, hbm_row)
      plsc.subcore_barrier()
      _combine(ua, None, d2, acc, abuf, obuf, scb, zcbuf, zbuf, 0, sub, True)
      return carry

    lax.fori_loop(0, k - 1, step, 0)
    plsc.subcore_barrier()
    _scatter_step(ua, CPT_M, acc, pball, gidx0, sidx0, gidx1, sidx1,
                  rb0, rb1, sem0, sem1, 0, hbm_row)
    plsc.subcore_barrier()
    _combine(out, None, dinv, acc, abuf, obuf, scb, zcbuf, zbuf, base, sub,
             True)

  chain(u_s1, 2, h0)
  chain(u_s2, 10, h1)
  chain(u_f, 2, z0)

  pltpu.sync_copy(pk_k.at[pl.ds(sub * EPT_K, EPT_K)],
                  pball.at[pl.ds(0, EPT_K)])
  plsc.subcore_barrier()
  _scatter_step(u_k, CPT_K, acc, pball, gidx0, sidx0, gidx1, sidx1,
                rb0, rb1, sem0, sem1, base, hbm_row)
  plsc.subcore_barrier()
  _combine(z1, z0, dinv_k, acc, abuf, obuf, scb, zcbuf, zbuf, base, sub,
           False)


@functools.cache
def _prop_kernel():
  return pl.kernel(
      _prop_body,
      out_type=(jax.ShapeDtypeStruct((NC * R, WS), jnp.float32),) * 4,
      mesh=_mesh(),
      scratch_types=[
          pltpu.VMEM_SHARED((R, WS), jnp.float32),
          pltpu.VMEM_SHARED((R, WS), jnp.float32),
          pltpu.VMEM((EPT_M,), jnp.int32),
          pltpu.VMEM((CH,), jnp.int32),
          pltpu.VMEM((CH,), jnp.int32),
          pltpu.VMEM((CH,), jnp.int32),
          pltpu.VMEM((CH,), jnp.int32),
          pltpu.VMEM((CH, WS), jnp.float32),
          pltpu.VMEM((CH, WS), jnp.float32),
          pltpu.VMEM((CCH, WS), jnp.float32),
          pltpu.VMEM((CCH, WS), jnp.float32),
          pltpu.VMEM((CCH, L), jnp.float32),
          pltpu.VMEM((CCH, WS), jnp.float32),
          pltpu.VMEM((CCH, WS), jnp.float32),
          pltpu.SemaphoreType.DMA,
          pltpu.SemaphoreType.DMA,
      ],
      compiler_params=pltpu.CompilerParams(use_tc_tiling_on_sc=False),
  )


def _deg_body(pk_m, pk_k, deg_m, deg_k, acc, pball, dball, onesb, zbuf, sem):
  core = lax.axis_index("c")
  sub = lax.axis_index("s")
  _fill(zbuf, CCH, L, 0.0)
  _fill(onesb, CH, L, 1.0)

  def histo(pk_ref, ept, deg_out):
    cpt = ept // CH
    nrow0 = sub * RPT
    pltpu.sync_copy(pk_ref.at[pl.ds(sub * ept, ept)],
                    pball.at[pl.ds(0, ept)])
    for i in range(RPT // CCH):
      pltpu.sync_copy(zbuf, acc.at[pl.ds(nrow0 + i * CCH, CCH)])

    def prep(i, c):
      for j in range(CH // L):
        w = pball[pl.ds(i * CH + j * L, L)]
        dball[i, pl.ds(j * L, L)] = w >> 16
      return c

    lax.fori_loop(0, cpt, prep, 0)
    plsc.subcore_barrier()

    def group(i, c):
      for j in range(8):
        pltpu.async_copy(onesb, acc.at[dball.at[8 * i + j]], sem, add=True)
      for j in range(8):
        pltpu.make_async_copy(deg_out.at[pl.ds(0, CH)], onesb, sem).wait()
      return c

    lax.fori_loop(0, cpt // 8, group, 0)
    for j in range(cpt % 8):
      pltpu.async_copy(onesb, acc.at[dball.at[8 * (cpt // 8) + j]], sem,
                       add=True)
    for j in range(cpt % 8):
      pltpu.make_async_copy(deg_out.at[pl.ds(0, CH)], onesb, sem).wait()
    plsc.subcore_barrier()

    def comb(i, c):
      r0 = nrow0 + i * CCH
      pltpu.sync_copy(acc.at[pl.ds(r0, CCH)], deg_out.at[pl.ds(r0, CCH)])
      return c

    lax.fori_loop(0, RPT // CCH, comb, 0)

  @pl.when(core == 0)
  def _():
    histo(pk_m, EPT_M, deg_m)

  @pl.when(core == 1)
  def _():
    histo(pk_k, EPT_K, deg_k)


@functools.cache
def _deg_kernel():
  return pl.kernel(
    _deg_body,
    out_type=(jax.ShapeDtypeStruct((R, L), jnp.float32),) * 2,
    mesh=_mesh(),
    scratch_types=[
        pltpu.VMEM_SHARED((R, L), jnp.float32),
        pltpu.VMEM((EPT_M,), jnp.int32),
        pltpu.VMEM((CPT_M, CH), jnp.int32),
        pltpu.VMEM((CH, L), jnp.float32),
        pltpu.VMEM((CCH, L), jnp.float32),
        pltpu.SemaphoreType.DMA,
    ],
    compiler_params=pltpu.CompilerParams(use_tc_tiling_on_sc=False),
  )


def _norm_body(dm_ref, dk_ref, im_ref, qm_ref, ik_ref):
  dm = dm_ref[...]
  im_ref[...] = lax.rsqrt(dm)
  qm_ref[...] = 1.0 / dm
  ik_ref[...] = lax.rsqrt(dk_ref[...])


def _norm(deg_m, deg_k):
  return pl.pallas_call(
      _norm_body,
      out_shape=(jax.ShapeDtypeStruct((R, L), jnp.float32),) * 3,
  )(deg_m, deg_k)


def _mm_body(x_ref, w_ref, b_ref, dm_ref, dk_ref, o_ref):
  y = jnp.dot(x_ref[...], w_ref[...], preferred_element_type=jnp.float32)
  y = y + b_ref[0:1, :]
  s = dm_ref[:, 0:1]
  sk = dk_ref[:, 0:1]
  o_ref[:, 0:384] = y[:, 0:384] * s
  o_ref[:, 384:512] = y[:, 384:512] * sk


_MMB = 1000


def _matmul(x, wt, b8, dm, dk):
  return pl.pallas_call(
      _mm_body,
      grid=(N // _MMB,),
      in_specs=[
          pl.BlockSpec((_MMB, D), lambda i: (i, 0)),
          pl.BlockSpec((D, 512), lambda i: (0, 0)),
          pl.BlockSpec((8, 512), lambda i: (0, 0)),
          pl.BlockSpec((_MMB, L), lambda i: (i, 0)),
          pl.BlockSpec((_MMB, L), lambda i: (i, 0)),
      ],
      out_specs=pl.BlockSpec((_MMB, 512), lambda i: (i, 0)),
      out_shape=jax.ShapeDtypeStruct((N, 512), jnp.float32),
  )(x, wt, b8, dm, dk)


def _pack_edges(ei, ept):
  e = ei.shape[1]
  tot = NS * ept
  src0 = ei[0].astype(jnp.int32)
  dst0 = ei[1].astype(jnp.int32)
  junk = DUM + (jnp.arange(e, dtype=jnp.int32) % (R - DUM))
  dst0 = jnp.where(src0 == dst0, junk, dst0)
  loop = jnp.arange(N, dtype=jnp.int32)
  npad = tot - e - N
  padj = DUM + (jnp.arange(npad, dtype=jnp.int32) % (R - DUM))
  src = jnp.concatenate([src0, loop, jnp.zeros((npad,), jnp.int32)])
  dst = jnp.concatenate([dst0, loop, padj])
  return src | (dst << 16)


def _to_sc(y):
  lo = jnp.pad(y[:, 0:WS], ((0, R - N), (0, 0)))
  hi = jnp.pad(y[:, WS:D], ((0, R - N), (0, 0)))
  return jnp.concatenate([lo, hi], axis=0)


def _from_sc(u):
  return jnp.concatenate([u[0:N], u[R:R + N]], axis=1)


def kernel(x, edge_index, knn_graph, W_s1, b_s1, W_s2, b_s2, W_f2, b_f2):
  pk_m = _pack_edges(edge_index, EPT_M)
  pk_k = _pack_edges(knn_graph, EPT_K)

  deg_m, deg_k = _deg_kernel()(pk_m, pk_k)
  dinv_m, d2_m, dinv_k = _norm(deg_m, deg_k)

  wt = jnp.concatenate([W_s1, W_s2, W_f2, W_f2], axis=0).T
  bcat = jnp.concatenate([b_s1, b_s2, b_f2, b_f2])
  b8 = jnp.broadcast_to(bcat[None, :], (8, 512))
  y = _matmul(x, wt, b8, dinv_m[0:N], dinv_k[0:N])

  h0u, h1u, z0u, z1u = _prop_kernel()(
      _to_sc(y[:, 0:128]), _to_sc(y[:, 128:256]), _to_sc(y[:, 256:384]),
      _to_sc(y[:, 384:512]), pk_m, pk_k, d2_m, dinv_m, dinv_k)

  return (_from_sc(h0u), _from_sc(h1u), _from_sc(z0u), _from_sc(z1u))
